# Initial kernel scaffold; baseline (speedup 1.0000x reference)
import jax
import jax.numpy as jnp
from jax.experimental import pallas as pl


def kernel(x, expert_weights, W_up, W_gate, W_down, W_pre, W_post, ln_g, ln_b, W_aproj, adapter_W, adapter_g, adapter_b, W_eproj, W_oproj):
    raise NotImplementedError("write your pallas kernel here")


# speedup vs baseline: 2.9084x; 2.9084x over previous
import functools

import jax
import jax.numpy as jnp
from jax.experimental import pallas as pl
from jax.experimental.pallas import tpu as pltpu

f32 = jnp.float32
bf16 = jnp.bfloat16


def _silu(v):
    return v * jax.nn.sigmoid(v)


def _ln(v, eps=1e-5):
    m = jnp.mean(v, axis=-1, keepdims=True)
    c = v - m
    var = jnp.mean(c * c, axis=-1, keepdims=True)
    return c * jax.lax.rsqrt(var + eps)


def _dot_t(a, b):
    return jax.lax.dot_general(a, b, (((1,), (1,)), ((), ())),
                               preferred_element_type=f32)


def _combine_body(wo_ref, we_ref, c_ref):
    c_ref[...] = jnp.dot(wo_ref[...], we_ref[...],
                         preferred_element_type=f32).astype(bf16)


def _pass1_body(E, x_ref, ew_ref, wup_ref, wgate_ref, wpre_ref, wpost_ref,
                lng_ref, lnb_ref, aw_ref, ag_ref, ab_ref,
                hid_ref, ain_ref, aout_ref, acc_ref):
    xb = x_ref[...].astype(bf16)
    up = _dot_t(xb, wup_ref[...])
    gate = _dot_t(xb, wgate_ref[...])
    hid = _silu(gate) * up
    hid_ref[...] = hid.astype(bf16)

    lng = lng_ref[...]
    lnb = lnb_ref[...]
    p = _dot_t(xb, wpre_ref[...])
    ain_ref[...] = (_ln(p) * lng + lnb).astype(bf16)
    ao = _dot_t(hid.astype(bf16), wpost_ref[...])
    aout_ref[...] = (_ln(ao) * lng + lnb).astype(bf16)

    pb = p.astype(bf16)
    w = ew_ref[...]
    coef = jnp.where(w > 0, w, 0.0)
    acc = jnp.zeros_like(p)
    for i in range(E):
        t = _dot_t(pb, aw_ref[i])
        t = _ln(t) * ag_ref[i:i + 1, :] + ab_ref[i:i + 1, :]
        acc = acc + coef[:, i:i + 1] * t
    acc_ref[...] = acc.astype(bf16)


def _pass2_body(ainb_ref, ainf_ref, aoutf_ref, hid_ref, acc_ref, ew_ref,
                waproj_ref, wdown_ref, c_ref, out_ref):
    qb = ainb_ref[0]
    scores = _dot_t(qb, aoutf_ref[0])
    sc = jnp.clip(scores, -5.0, 5.0)
    aw = _silu(sc)
    adapt = jnp.dot(aw.astype(bf16), ainf_ref[0], preferred_element_type=f32)
    hid = hid_ref[...].astype(f32)
    hid = hid + 0.1 * _dot_t(adapt.astype(bf16), waproj_ref[...])
    shared = _dot_t(hid.astype(bf16), wdown_ref[...])
    wsum = jnp.sum(ew_ref[...], axis=1, keepdims=True)
    eout = _dot_t(acc_ref[...], c_ref[...])
    out_ref[...] = shared * wsum + 0.1 * eout


def kernel(x, expert_weights, W_up, W_gate, W_down, W_pre, W_post, ln_g, ln_b,
           W_aproj, adapter_W, adapter_g, adapter_b, W_eproj, W_oproj):
    B, S, D = x.shape
    E = expert_weights.shape[-1]
    H = W_up.shape[0]
    A = W_pre.shape[0]
    N = B * S
    BT1 = 512
    BT2 = 512
    NSB = S // BT2

    xt = x.reshape(N, D)
    ew = expert_weights.reshape(N, E)
    lng = ln_g.reshape(1, A).astype(f32)
    lnb = ln_b.reshape(1, A).astype(f32)

    C = pl.pallas_call(
        _combine_body,
        out_shape=jax.ShapeDtypeStruct((D, A), bf16),
    )(W_oproj.astype(bf16), W_eproj.astype(bf16))

    full = lambda shape: pl.BlockSpec(shape, lambda i: (0,) * len(shape))
    hidden, ain, aout, acc = pl.pallas_call(
        functools.partial(_pass1_body, E),
        grid=(N // BT1,),
        in_specs=[
            pl.BlockSpec((BT1, D), lambda i: (i, 0)),
            pl.BlockSpec((BT1, E), lambda i: (i, 0)),
            full((H, D)),
            full((H, D)),
            full((A, D)),
            full((A, H)),
            full((1, A)),
            full((1, A)),
            full((E, A, A)),
            full((E, A)),
            full((E, A)),
        ],
        out_specs=[
            pl.BlockSpec((BT1, H), lambda i: (i, 0)),
            pl.BlockSpec((BT1, A), lambda i: (i, 0)),
            pl.BlockSpec((BT1, A), lambda i: (i, 0)),
            pl.BlockSpec((BT1, A), lambda i: (i, 0)),
        ],
        out_shape=[
            jax.ShapeDtypeStruct((N, H), bf16),
            jax.ShapeDtypeStruct((N, A), bf16),
            jax.ShapeDtypeStruct((N, A), bf16),
            jax.ShapeDtypeStruct((N, A), bf16),
        ],
        compiler_params=pltpu.CompilerParams(
            dimension_semantics=("parallel",)),
    )(xt, ew, W_up.astype(bf16), W_gate.astype(bf16), W_pre.astype(bf16),
      W_post.astype(bf16), lng, lnb, adapter_W.astype(bf16),
      adapter_g.astype(f32), adapter_b.astype(f32))

    ain3 = ain.reshape(B, S, A)
    aout3 = aout.reshape(B, S, A)
    tok = lambda b, j: (b * NSB + j, 0)
    out = pl.pallas_call(
        _pass2_body,
        grid=(B, NSB),
        in_specs=[
            pl.BlockSpec((1, BT2, A), lambda b, j: (b, j, 0)),
            pl.BlockSpec((1, S, A), lambda b, j: (b, 0, 0)),
            pl.BlockSpec((1, S, A), lambda b, j: (b, 0, 0)),
            pl.BlockSpec((BT2, H), tok),
            pl.BlockSpec((BT2, A), tok),
            pl.BlockSpec((BT2, E), tok),
            pl.BlockSpec((H, A), lambda b, j: (0, 0)),
            pl.BlockSpec((D, H), lambda b, j: (0, 0)),
            pl.BlockSpec((D, A), lambda b, j: (0, 0)),
        ],
        out_specs=pl.BlockSpec((BT2, D), tok),
        out_shape=jax.ShapeDtypeStruct((N, D), f32),
        compiler_params=pltpu.CompilerParams(
            dimension_semantics=("arbitrary", "arbitrary")),
    )(ain3, ain3, aout3, hidden, acc, ew,
      W_aproj.astype(bf16), W_down.astype(bf16), C)

    return out.reshape(B, S, D)


# speedup vs baseline: 3.2763x; 1.1265x over previous
import functools

import jax
import jax.numpy as jnp
from jax.experimental import pallas as pl
from jax.experimental.pallas import tpu as pltpu

f32 = jnp.float32
bf16 = jnp.bfloat16


def _silu(v):
    return v * jax.nn.sigmoid(v)


def _ln(v, eps=1e-5):
    m = jnp.mean(v, axis=-1, keepdims=True)
    c = v - m
    var = jnp.mean(c * c, axis=-1, keepdims=True)
    return c * jax.lax.rsqrt(var + eps)


def _dot_t(a, b):
    return jax.lax.dot_general(a, b, (((1,), (1,)), ((), ())),
                               preferred_element_type=f32)


def _combine_body(wo_ref, we_ref, c_ref):
    c_ref[...] = jnp.dot(wo_ref[...].astype(bf16), we_ref[...].astype(bf16),
                         preferred_element_type=f32).astype(bf16)


def _pass1_body(E, x_ref, ew_ref, wup_ref, wgate_ref, wpre_ref, wpost_ref,
                lng_ref, lnb_ref, aw_ref, ag_ref, ab_ref,
                hid_ref, ain_ref, aout_ref, acc_ref, wub_ref, wgb_ref):
    @pl.when(pl.program_id(0) == 0)
    def _cast_weights():
        wub_ref[...] = wup_ref[...].astype(bf16)
        wgb_ref[...] = wgate_ref[...].astype(bf16)

    xb = x_ref[...].astype(bf16)
    up = _dot_t(xb, wub_ref[...])
    gate = _dot_t(xb, wgb_ref[...])
    hid = _silu(gate) * up
    hid_ref[...] = hid.astype(bf16)

    lng = lng_ref[...]
    lnb = lnb_ref[...]
    p = _dot_t(xb, wpre_ref[...])
    ain_ref[...] = (_ln(p) * lng + lnb).astype(bf16)
    ao = _dot_t(hid.astype(bf16), wpost_ref[...])
    aout_ref[...] = (_ln(ao) * lng + lnb).astype(bf16)

    pb = p.astype(bf16)
    w = ew_ref[...]
    coef = jnp.where(w > 0, w, 0.0)
    acc = jnp.zeros_like(p)
    for i in range(E):
        t = _dot_t(pb, aw_ref[i])
        t = _ln(t) * ag_ref[i:i + 1, :] + ab_ref[i:i + 1, :]
        acc = acc + coef[:, i:i + 1] * t
    acc_ref[...] = acc.astype(bf16)


def _pass2_body(ainb_ref, ainf_ref, aoutf_ref, hid_ref, acc_ref, ew_ref,
                waproj_ref, wdown_ref, c_ref, out_ref, wdb_ref):
    @pl.when(jnp.logical_and(pl.program_id(0) == 0, pl.program_id(1) == 0))
    def _cast_weights():
        wdb_ref[...] = wdown_ref[...].astype(bf16)

    qb = ainb_ref[0]
    scores = _dot_t(qb, aoutf_ref[0])
    sc = jnp.clip(scores, -5.0, 5.0)
    aw = _silu(sc)
    adapt = jnp.dot(aw.astype(bf16), ainf_ref[0], preferred_element_type=f32)
    hid = hid_ref[...].astype(f32)
    hid = hid + 0.1 * _dot_t(adapt.astype(bf16), waproj_ref[...])
    shared = _dot_t(hid.astype(bf16), wdb_ref[...])
    wsum = jnp.sum(ew_ref[...], axis=1, keepdims=True)
    eout = _dot_t(acc_ref[...], c_ref[...])
    out_ref[...] = shared * wsum + 0.1 * eout


def kernel(x, expert_weights, W_up, W_gate, W_down, W_pre, W_post, ln_g, ln_b,
           W_aproj, adapter_W, adapter_g, adapter_b, W_eproj, W_oproj):
    B, S, D = x.shape
    E = expert_weights.shape[-1]
    H = W_up.shape[0]
    A = W_pre.shape[0]
    N = B * S
    BT1 = 512
    BT2 = 512
    NSB = S // BT2

    xt = x.reshape(N, D)
    ew = expert_weights.reshape(N, E)
    lng = ln_g.reshape(1, A).astype(f32)
    lnb = ln_b.reshape(1, A).astype(f32)

    C = pl.pallas_call(
        _combine_body,
        out_shape=jax.ShapeDtypeStruct((D, A), bf16),
    )(W_oproj, W_eproj)

    full = lambda shape: pl.BlockSpec(shape, lambda i: (0,) * len(shape))
    hidden, ain, aout, acc = pl.pallas_call(
        functools.partial(_pass1_body, E),
        grid=(N // BT1,),
        in_specs=[
            pl.BlockSpec((BT1, D), lambda i: (i, 0)),
            pl.BlockSpec((BT1, E), lambda i: (i, 0)),
            full((H, D)),
            full((H, D)),
            full((A, D)),
            full((A, H)),
            full((1, A)),
            full((1, A)),
            full((E, A, A)),
            full((E, A)),
            full((E, A)),
        ],
        out_specs=[
            pl.BlockSpec((BT1, H), lambda i: (i, 0)),
            pl.BlockSpec((BT1, A), lambda i: (i, 0)),
            pl.BlockSpec((BT1, A), lambda i: (i, 0)),
            pl.BlockSpec((BT1, A), lambda i: (i, 0)),
        ],
        out_shape=[
            jax.ShapeDtypeStruct((N, H), bf16),
            jax.ShapeDtypeStruct((N, A), bf16),
            jax.ShapeDtypeStruct((N, A), bf16),
            jax.ShapeDtypeStruct((N, A), bf16),
        ],
        scratch_shapes=[
            pltpu.VMEM((H, D), bf16),
            pltpu.VMEM((H, D), bf16),
        ],
        compiler_params=pltpu.CompilerParams(
            dimension_semantics=("arbitrary",)),
    )(xt, ew, W_up, W_gate, W_pre.astype(bf16),
      W_post.astype(bf16), lng, lnb, adapter_W.astype(bf16),
      adapter_g.astype(f32), adapter_b.astype(f32))

    ain3 = ain.reshape(B, S, A)
    aout3 = aout.reshape(B, S, A)
    tok = lambda b, j: (b * NSB + j, 0)
    out = pl.pallas_call(
        _pass2_body,
        grid=(B, NSB),
        in_specs=[
            pl.BlockSpec((1, BT2, A), lambda b, j: (b, j, 0)),
            pl.BlockSpec((1, S, A), lambda b, j: (b, 0, 0)),
            pl.BlockSpec((1, S, A), lambda b, j: (b, 0, 0)),
            pl.BlockSpec((BT2, H), tok),
            pl.BlockSpec((BT2, A), tok),
            pl.BlockSpec((BT2, E), tok),
            pl.BlockSpec((H, A), lambda b, j: (0, 0)),
            pl.BlockSpec((D, H), lambda b, j: (0, 0)),
            pl.BlockSpec((D, A), lambda b, j: (0, 0)),
        ],
        out_specs=pl.BlockSpec((BT2, D), tok),
        out_shape=jax.ShapeDtypeStruct((N, D), f32),
        scratch_shapes=[pltpu.VMEM((D, H), bf16)],
        compiler_params=pltpu.CompilerParams(
            dimension_semantics=("arbitrary", "arbitrary")),
    )(ain3, ain3, aout3, hidden, acc, ew,
      W_aproj.astype(bf16), W_down, C)

    return out.reshape(B, S, D)
